# baseline (speedup 1.0000x reference)
import functools

import jax
import jax.numpy as jnp
from jax import lax
from jax.experimental import pallas as pl
from jax.experimental.pallas import tpu as pltpu
from jax.experimental.pallas import tpu_sc as plsc

NC = 2
NS = 16
L = 16

B, CH, Q, D = 64, 2, 32, 2048
NB = 29
NBP = 32
HSZ = L * NBP
QC = 8
NQC = Q // QC
B_PER_W = B // (NC * NS)


NCHUNK = B_PER_W * NQC


def _sc_body(sim_hbm, w_hbm, out_hbm, sim_buf, w_buf, hist, out_stage,
             sem0, sem1):
    wid = lax.axis_index("s") * NC + lax.axis_index("c")
    lane = lax.broadcasted_iota(jnp.int32, (L,), 0)
    lane_adj0 = lane * NBP - 0x4B000000
    lane_adj1 = lane_adj0 + HSZ
    lane_div4 = lane >> 2
    shamt = (lane & 3) * 8
    zeros16 = jnp.zeros((L,), jnp.float32)
    sems = (sem0, sem1)

    for v in range((CH * HSZ) // L):
        hist[pl.ds(v * L, L)] = zeros16

    def bq(t):
        b_off = t // NQC
        qc = t - b_off * NQC
        return wid * B_PER_W + b_off, qc

    def issue(t, p):
        b, qc = bq(t)
        pltpu.async_copy(w_hbm.at[b, pl.ds(qc * QC, QC)],
                         w_buf.at[p], sems[p])
        pltpu.async_copy(sim_hbm.at[b, 0, pl.ds(qc * QC, QC)],
                         sim_buf.at[p, 0], sems[p])
        pltpu.async_copy(sim_hbm.at[b, 1, pl.ds(qc * QC, QC)],
                         sim_buf.at[p, 1], sems[p])

    def wait(t, p):
        b, qc = bq(t)
        pltpu.make_async_copy(w_hbm.at[b, pl.ds(qc * QC, QC)],
                              w_buf.at[p], sems[p]).wait()
        pltpu.make_async_copy(sim_hbm.at[b, 0, pl.ds(qc * QC, QC)],
                              sim_buf.at[p, 0], sems[p]).wait()
        pltpu.make_async_copy(sim_hbm.at[b, 1, pl.ds(qc * QC, QC)],
                              sim_buf.at[p, 1], sems[p]).wait()

    def compute(t, p):
        b, qc = bq(t)

        def q_body(q, _, p=p, qc=qc):
            qvec = q + jnp.zeros((L,), jnp.int32)

            @plsc.parallel_loop(0, D // (4 * L), unroll=2)
            def _(g, q=q, p=p, qvec=qvec):
                gw = g * L
                for j in range(4):
                    mw = plsc.load_gather(
                        w_buf.at[p], [qvec, gw + (lane_div4 + 4 * j)])
                    wv = ((mw >> shamt) & 1).astype(jnp.float32)
                    off = g * (4 * L) + j * L
                    s0 = sim_buf[p, 0, q, pl.ds(off, L)]
                    s1 = sim_buf[p, 1, q, pl.ds(off, L)]
                    u0 = (s0 * 14.0 + 13.50014) + 8388608.0
                    u1 = (s1 * 14.0 + 13.50014) + 8388608.0
                    plsc.addupdate_scatter(
                        hist, [plsc.bitcast(u0, jnp.int32) + lane_adj0], wv)
                    plsc.addupdate_scatter(
                        hist, [plsc.bitcast(u1, jnp.int32) + lane_adj1], wv)

            qg = qc * QC + q
            for ch in range(CH):
                base = ch * HSZ
                lo = hist[pl.ds(base, L)]
                hi = hist[pl.ds(base + L, L)]
                hist[pl.ds(base, L)] = zeros16
                hist[pl.ds(base + L, L)] = zeros16
                for l in range(1, L):
                    o = base + l * NBP
                    lo = lo + hist[pl.ds(o, L)]
                    hi = hi + hist[pl.ds(o + L, L)]
                    hist[pl.ds(o, L)] = zeros16
                    hist[pl.ds(o + L, L)] = zeros16
                out_stage[ch, qg, pl.ds(0, L)] = lo
                out_stage[ch, qg, pl.ds(L, L)] = hi
            return 0

        lax.fori_loop(0, QC, q_body, 0)
        @pl.when(qc == NQC - 1)
        def _():
            pltpu.sync_copy(out_stage, out_hbm.at[b])

    issue(0, 0)
    def pair_body(tp, _):
        t0 = 2 * tp
        issue(t0 + 1, 1)
        wait(t0, 0)
        compute(t0, 0)

        @pl.when(tp < NCHUNK // 2 - 1)
        def _():
            issue(t0 + 2, 0)

        wait(t0 + 1, 1)
        compute(t0 + 1, 1)
        return 0

    lax.fori_loop(0, NCHUNK // 2, pair_body, 0)


_hist_kernel = functools.partial(
    pl.kernel,
    mesh=plsc.VectorSubcoreMesh(core_axis_name="c", subcore_axis_name="s",
                                num_cores=NC, num_subcores=NS),
    out_type=jax.ShapeDtypeStruct((B, CH, Q, NBP), jnp.float32),
    scratch_types=[
        pltpu.VMEM((2, CH, QC, D), jnp.float32),
        pltpu.VMEM((2, QC, D // 4), jnp.int32),
        pltpu.VMEM((CH * HSZ,), jnp.float32),
        pltpu.VMEM((CH, Q, NBP), jnp.float32),
        pltpu.SemaphoreType.DMA,
        pltpu.SemaphoreType.DMA,
    ],
    compiler_params=pltpu.CompilerParams(needs_layout_passes=False),
)(_sc_body)


def kernel(simmat, dlens, mask):
    del dlens
    w32 = lax.bitcast_convert_type(
        mask.astype(jnp.uint8).reshape(B, Q, D // 4, 4), jnp.int32)
    out_pad = _hist_kernel(simmat, w32)
    return out_pad[..., :NB]


# speedup vs baseline: 1.0347x; 1.0347x over previous
import functools

import jax
import jax.numpy as jnp
from jax import lax
from jax.experimental import pallas as pl
from jax.experimental.pallas import tpu as pltpu
from jax.experimental.pallas import tpu_sc as plsc

NC = 2
NS = 16
L = 16

B, CH, Q, D = 64, 2, 32, 2048
NB = 29
NBP = 32
HSZ = L * NBP
QC = 8
NQC = Q // QC
B_PER_W = B // (NC * NS)


NCHUNK = B_PER_W * NQC


def _vperm(x, idx):
    return lax.gather(
        x, idx[:, None],
        dimension_numbers=lax.GatherDimensionNumbers(
            offset_dims=(), collapsed_slice_dims=(0,), start_index_map=(0,)),
        slice_sizes=(1,),
        mode=lax.GatherScatterMode.PROMISE_IN_BOUNDS)


def _sc_body(sim_hbm, w_hbm, out_hbm, sim_buf, w_buf, hist, out_stage,
             sem0, sem1):
    wid = lax.axis_index("s") * NC + lax.axis_index("c")
    lane = lax.broadcasted_iota(jnp.int32, (L,), 0)
    lane_adj0 = lane * NBP - 0x4B000000
    lane_adj1 = lane_adj0 + HSZ
    lane_div4 = lane >> 2
    shamt = (lane & 3) * 8
    zeros16 = jnp.zeros((L,), jnp.float32)
    sems = (sem0, sem1)

    for v in range((CH * HSZ) // L):
        hist[pl.ds(v * L, L)] = zeros16

    def bq(t):
        b_off = t // NQC
        qc = t - b_off * NQC
        return wid * B_PER_W + b_off, qc

    def issue(t, p):
        b, qc = bq(t)
        pltpu.async_copy(w_hbm.at[b, pl.ds(qc * QC, QC)],
                         w_buf.at[p], sems[p])
        pltpu.async_copy(sim_hbm.at[b, 0, pl.ds(qc * QC, QC)],
                         sim_buf.at[p, 0], sems[p])
        pltpu.async_copy(sim_hbm.at[b, 1, pl.ds(qc * QC, QC)],
                         sim_buf.at[p, 1], sems[p])

    def wait(t, p):
        b, qc = bq(t)
        pltpu.make_async_copy(w_hbm.at[b, pl.ds(qc * QC, QC)],
                              w_buf.at[p], sems[p]).wait()
        pltpu.make_async_copy(sim_hbm.at[b, 0, pl.ds(qc * QC, QC)],
                              sim_buf.at[p, 0], sems[p]).wait()
        pltpu.make_async_copy(sim_hbm.at[b, 1, pl.ds(qc * QC, QC)],
                              sim_buf.at[p, 1], sems[p]).wait()

    def compute(t, p):
        b, qc = bq(t)

        def q_body(q, _, p=p, qc=qc):
            @plsc.parallel_loop(0, D // (4 * L), unroll=2)
            def _(g, q=q, p=p):
                mw_all = w_buf[p, q, pl.ds(g * L, L)]
                for j in range(4):
                    mw = _vperm(mw_all, lane_div4 + 4 * j)
                    wv = ((mw >> shamt) & 1).astype(jnp.float32)
                    off = g * (4 * L) + j * L
                    s0 = sim_buf[p, 0, q, pl.ds(off, L)]
                    s1 = sim_buf[p, 1, q, pl.ds(off, L)]
                    u0 = (s0 * 14.0 + 13.50014) + 8388608.0
                    u1 = (s1 * 14.0 + 13.50014) + 8388608.0
                    plsc.addupdate_scatter(
                        hist, [plsc.bitcast(u0, jnp.int32) + lane_adj0], wv)
                    plsc.addupdate_scatter(
                        hist, [plsc.bitcast(u1, jnp.int32) + lane_adj1], wv)

            qg = qc * QC + q
            for ch in range(CH):
                base = ch * HSZ
                lo = hist[pl.ds(base, L)]
                hi = hist[pl.ds(base + L, L)]
                hist[pl.ds(base, L)] = zeros16
                hist[pl.ds(base + L, L)] = zeros16
                for l in range(1, L):
                    o = base + l * NBP
                    lo = lo + hist[pl.ds(o, L)]
                    hi = hi + hist[pl.ds(o + L, L)]
                    hist[pl.ds(o, L)] = zeros16
                    hist[pl.ds(o + L, L)] = zeros16
                out_stage[ch, qg, pl.ds(0, L)] = lo
                out_stage[ch, qg, pl.ds(L, L)] = hi
            return 0

        lax.fori_loop(0, QC, q_body, 0)
        @pl.when(qc == NQC - 1)
        def _():
            pltpu.sync_copy(out_stage, out_hbm.at[b])

    issue(0, 0)
    def pair_body(tp, _):
        t0 = 2 * tp
        issue(t0 + 1, 1)
        wait(t0, 0)
        compute(t0, 0)

        @pl.when(tp < NCHUNK // 2 - 1)
        def _():
            issue(t0 + 2, 0)

        wait(t0 + 1, 1)
        compute(t0 + 1, 1)
        return 0

    lax.fori_loop(0, NCHUNK // 2, pair_body, 0)


_hist_kernel = functools.partial(
    pl.kernel,
    mesh=plsc.VectorSubcoreMesh(core_axis_name="c", subcore_axis_name="s",
                                num_cores=NC, num_subcores=NS),
    out_type=jax.ShapeDtypeStruct((B, CH, Q, NBP), jnp.float32),
    scratch_types=[
        pltpu.VMEM((2, CH, QC, D), jnp.float32),
        pltpu.VMEM((2, QC, D // 4), jnp.int32),
        pltpu.VMEM((CH * HSZ,), jnp.float32),
        pltpu.VMEM((CH, Q, NBP), jnp.float32),
        pltpu.SemaphoreType.DMA,
        pltpu.SemaphoreType.DMA,
    ],
    compiler_params=pltpu.CompilerParams(needs_layout_passes=False),
)(_sc_body)


def kernel(simmat, dlens, mask):
    del dlens
    w32 = lax.bitcast_convert_type(
        mask.astype(jnp.uint8).reshape(B, Q, D // 4, 4), jnp.int32)
    out_pad = _hist_kernel(simmat, w32)
    return out_pad[..., :NB]


# speedup vs baseline: 1.5303x; 1.4789x over previous
import functools

import jax
import jax.numpy as jnp
from jax import lax
from jax.experimental import pallas as pl
from jax.experimental.pallas import tpu as pltpu
from jax.experimental.pallas import tpu_sc as plsc

NC = 2
NS = 16
L = 16

B, CH, Q, D = 64, 2, 32, 2048
NB = 29
NBP = 32
HSZ = L * NBP
QC = 8
NQC = Q // QC
B_PER_W = B // (NC * NS)


NCHUNK = B_PER_W * NQC


def _vperm(x, idx):
    return lax.gather(
        x, idx[:, None],
        dimension_numbers=lax.GatherDimensionNumbers(
            offset_dims=(), collapsed_slice_dims=(0,), start_index_map=(0,)),
        slice_sizes=(1,),
        mode=lax.GatherScatterMode.PROMISE_IN_BOUNDS)


def _sc_body(sim_hbm, w_hbm, out_hbm, sim_buf, w_buf, hist, out_stage,
             sem0, sem1):
    wid = lax.axis_index("s") * NC + lax.axis_index("c")
    lane = lax.broadcasted_iota(jnp.int32, (L,), 0)
    lane_adj0 = lane * NBP - 0x4B000000
    lane_adj1 = lane_adj0 + HSZ
    lane_div4 = lane >> 2
    shamt = (lane & 3) * 8
    zeros16 = jnp.zeros((L,), jnp.float32)
    sems = (sem0, sem1)

    for v in range((CH * HSZ) // L):
        hist[pl.ds(v * L, L)] = zeros16

    def bq(t):
        b_off = t // NQC
        qc = t - b_off * NQC
        return wid * B_PER_W + b_off, qc

    def issue(t, p):
        b, qc = bq(t)
        pltpu.async_copy(w_hbm.at[b, pl.ds(qc * QC, QC)],
                         w_buf.at[p], sems[p])
        pltpu.async_copy(sim_hbm.at[b, 0, pl.ds(qc * QC, QC)],
                         sim_buf.at[p, 0], sems[p])
        pltpu.async_copy(sim_hbm.at[b, 1, pl.ds(qc * QC, QC)],
                         sim_buf.at[p, 1], sems[p])

    def wait(t, p):
        b, qc = bq(t)
        pltpu.make_async_copy(w_hbm.at[b, pl.ds(qc * QC, QC)],
                              w_buf.at[p], sems[p]).wait()
        pltpu.make_async_copy(sim_hbm.at[b, 0, pl.ds(qc * QC, QC)],
                              sim_buf.at[p, 0], sems[p]).wait()
        pltpu.make_async_copy(sim_hbm.at[b, 1, pl.ds(qc * QC, QC)],
                              sim_buf.at[p, 1], sems[p]).wait()

    def compute(t, p):
        b, qc = bq(t)

        def q_body(q, _, p=p, qc=qc):
            @plsc.parallel_loop(0, D // L, unroll=8)
            def _(i, q=q, p=p):
                off = i * L
                wv = w_buf[p, q, pl.ds(off, L)]
                s0 = sim_buf[p, 0, q, pl.ds(off, L)]
                s1 = sim_buf[p, 1, q, pl.ds(off, L)]
                u0 = (s0 * 14.0 + 13.50014) + 8388608.0
                u1 = (s1 * 14.0 + 13.50014) + 8388608.0
                plsc.addupdate_scatter(
                    hist, [plsc.bitcast(u0, jnp.int32) + lane_adj0], wv)
                plsc.addupdate_scatter(
                    hist, [plsc.bitcast(u1, jnp.int32) + lane_adj1], wv)

            qg = qc * QC + q
            for ch in range(CH):
                base = ch * HSZ
                lo = hist[pl.ds(base, L)]
                hi = hist[pl.ds(base + L, L)]
                hist[pl.ds(base, L)] = zeros16
                hist[pl.ds(base + L, L)] = zeros16
                for l in range(1, L):
                    o = base + l * NBP
                    lo = lo + hist[pl.ds(o, L)]
                    hi = hi + hist[pl.ds(o + L, L)]
                    hist[pl.ds(o, L)] = zeros16
                    hist[pl.ds(o + L, L)] = zeros16
                out_stage[ch, qg, pl.ds(0, L)] = lo
                out_stage[ch, qg, pl.ds(L, L)] = hi
            return 0

        lax.fori_loop(0, QC, q_body, 0)
        @pl.when(qc == NQC - 1)
        def _():
            pltpu.sync_copy(out_stage, out_hbm.at[b])

    issue(0, 0)
    def pair_body(tp, _):
        t0 = 2 * tp
        issue(t0 + 1, 1)
        wait(t0, 0)
        compute(t0, 0)

        @pl.when(tp < NCHUNK // 2 - 1)
        def _():
            issue(t0 + 2, 0)

        wait(t0 + 1, 1)
        compute(t0 + 1, 1)
        return 0

    lax.fori_loop(0, NCHUNK // 2, pair_body, 0)


_hist_kernel = functools.partial(
    pl.kernel,
    mesh=plsc.VectorSubcoreMesh(core_axis_name="c", subcore_axis_name="s",
                                num_cores=NC, num_subcores=NS),
    out_type=jax.ShapeDtypeStruct((B, CH, Q, NBP), jnp.float32),
    scratch_types=[
        pltpu.VMEM((2, CH, QC, D), jnp.float32),
        pltpu.VMEM((2, QC, D), jnp.float32),
        pltpu.VMEM((CH * HSZ,), jnp.float32),
        pltpu.VMEM((CH, Q, NBP), jnp.float32),
        pltpu.SemaphoreType.DMA,
        pltpu.SemaphoreType.DMA,
    ],
    compiler_params=pltpu.CompilerParams(needs_layout_passes=False),
)(_sc_body)


def kernel(simmat, dlens, mask):
    del dlens
    out_pad = _hist_kernel(simmat, mask.astype(jnp.float32))
    return out_pad[..., :NB]
